# Initial kernel scaffold; baseline (speedup 1.0000x reference)
import jax
import jax.numpy as jnp
from jax.experimental import pallas as pl


def kernel(points, features, bn_fts_g, bn_fts_b, ec1_W0, ec1_W1, ec1_W2, ec1_g0, ec1_b0, ec1_g1, ec1_b1, ec1_g2, ec1_b2, ec2_W0, ec2_W1, ec2_W2, ec2_g0, ec2_b0, ec2_g1, ec2_b1, ec2_g2, ec2_b2, ec2_Wsc, ec2_sc_g, ec2_sc_b, fus_W, fus_g, fus_b, fc1_W, fc1_b, fc2_W, fc2_b):
    raise NotImplementedError("write your pallas kernel here")


# speedup vs baseline: 3.0192x; 3.0192x over previous
import jax
import jax.numpy as jnp
from jax.experimental import pallas as pl
from jax.experimental.pallas import tpu as pltpu

B = 256
N = 128
K = 7
C1 = 32
C2 = 64
BB = 8
NB = B // BB
NK = N * K
EPS = 1e-5
NEG = -3.0e38
CNTF = float(B * N)
CNTK = float(B * N * K)


def _bn_params(s, ss, cnt, g, b):
    m = s / cnt
    v = ss / cnt - m * m
    scale = g * jax.lax.rsqrt(v + EPS)
    shift = b - m * scale
    return scale, shift


def _mask_of(f):
    return (jnp.sum(jnp.abs(f), axis=1, keepdims=True) != 0.0).astype(jnp.float32)


def _knn_idx(pts):
    rows = []
    for i in range(BB):
        p = pts[i]
        pt = p.T
        g = jax.lax.dot_general(pt, p, (((1,), (0,)), ((), ())),
                                preferred_element_type=jnp.float32)
        xx = jnp.sum(pt * pt, axis=1, keepdims=True)
        rows.append(2.0 * g - xx - xx.T)
    pd = jnp.concatenate(rows, axis=0)
    lane = jax.lax.broadcasted_iota(jnp.int32, (BB * N, N), 1)
    cols = []
    for _ in range(K + 1):
        m = jnp.max(pd, axis=1, keepdims=True)
        cand = jnp.where(pd >= m, lane, N)
        j = jnp.min(cand, axis=1, keepdims=True)
        cols.append(j)
        pd = jnp.where(lane == j, NEG, pd)
    return jnp.concatenate(cols, axis=1)


def _gather_edges(ftsT, idx7):
    C = ftsT.shape[-1]
    lane = jax.lax.broadcasted_iota(jnp.int32, (N, K, N), 2)
    outs = []
    for i in range(BB):
        idxi = idx7[i * N:(i + 1) * N, :]
        oh = (idxi[:, :, None] == lane).astype(jnp.float32)
        nb = jax.lax.dot_general(oh.reshape(NK, N), ftsT[i],
                                 (((1,), (0,)), ((), ())),
                                 preferred_element_type=jnp.float32)
        outs.append(nb)
    nbr = jnp.concatenate(outs, axis=0)
    ctr = jnp.broadcast_to(ftsT[:, :, None, :], (BB, N, K, C)).reshape(BB * NK, C)
    return jnp.concatenate([ctr, nbr - ctr], axis=1)


def _mm(x, w):
    return jax.lax.dot_general(x, w, (((1,), (1,)), ((), ())),
                               preferred_element_type=jnp.float32)


def _acc(ref, row, y):
    ref[row, :] += jnp.sum(y, axis=0)
    ref[row + 1, :] += jnp.sum(y * y, axis=0)


def _ec1_fn(f, mask, acc, g0, b0):
    sc0, sh0 = _bn_params(acc[0, :], acc[1, :], CNTF, g0[0, :], b0[0, :])
    return (f * sc0[None, :, None] + sh0[None, :, None]) * mask


def _ec1_stats(points_ref, feat_ref, g0_ref, b0_ref, w_ref, gb_ref,
               acc_ref, idx_s):
    s = pl.program_id(0)
    blk = pl.program_id(1)

    @pl.when(jnp.logical_and(s == 0, blk == 0))
    def _zero():
        acc_ref[...] = jnp.zeros_like(acc_ref)

    f = feat_ref[...]
    mask = _mask_of(f)

    @pl.when(s == 0)
    def _s0():
        pts = points_ref[...] * mask + (1.0 - mask) * 1e9
        idx_s[:, pl.ds(blk * BB * N, BB * N)] = _knn_idx(pts).T
        acc_ref[0, :] += jnp.sum(f, axis=(0, 2))
        acc_ref[1, :] += jnp.sum(f * f, axis=(0, 2))

    @pl.when(s > 0)
    def _rest():
        fn = _ec1_fn(f, mask, acc_ref, g0_ref, b0_ref)
        ftsT = jnp.transpose(fn, (0, 2, 1))
        idx8 = idx_s[:, pl.ds(blk * BB * N, BB * N)].T
        y = _mm(_gather_edges(ftsT, idx8[:, 1:]), w_ref[0:32, 0:64])

        @pl.when(s == 1)
        def _():
            _acc(acc_ref, 2, y)

        @pl.when(s > 1)
        def _d1():
            sc, sh = _bn_params(acc_ref[2, :], acc_ref[3, :], CNTK,
                                gb_ref[0, :], gb_ref[1, :])
            y2 = _mm(jnp.maximum(y * sc + sh, 0.0), w_ref[32:64, 0:32])

            @pl.when(s == 2)
            def _():
                _acc(acc_ref, 4, y2)

            @pl.when(s == 3)
            def _d2():
                sc2, sh2 = _bn_params(acc_ref[4, :], acc_ref[5, :], CNTK,
                                      gb_ref[2, :], gb_ref[3, :])
                y3 = _mm(jnp.maximum(y2 * sc2 + sh2, 0.0), w_ref[64:96, 0:32])
                _acc(acc_ref, 6, y3)


def _ec1_apply(points_ref, feat_ref, g0_ref, b0_ref, w_ref, gb_ref, acc_ref,
               out_ref):
    f = feat_ref[...]
    mask = _mask_of(f)
    pts = points_ref[...] * mask + (1.0 - mask) * 1e9
    idx8 = _knn_idx(pts)
    fn = _ec1_fn(f, mask, acc_ref, g0_ref, b0_ref)
    ftsT = jnp.transpose(fn, (0, 2, 1))
    y = _mm(_gather_edges(ftsT, idx8[:, 1:]), w_ref[0:32, 0:64])
    sc, sh = _bn_params(acc_ref[2, :], acc_ref[3, :], CNTK,
                        gb_ref[0, :], gb_ref[1, :])
    y2 = _mm(jnp.maximum(y * sc + sh, 0.0), w_ref[32:64, 0:32])
    sc2, sh2 = _bn_params(acc_ref[4, :], acc_ref[5, :], CNTK,
                          gb_ref[2, :], gb_ref[3, :])
    y3 = _mm(jnp.maximum(y2 * sc2 + sh2, 0.0), w_ref[64:96, 0:32])
    sc3, sh3 = _bn_params(acc_ref[6, :], acc_ref[7, :], CNTK,
                          gb_ref[4, :], gb_ref[5, :])
    z3 = jnp.maximum(y3 * sc3 + sh3, 0.0)
    zm = jnp.mean(z3.reshape(BB, N, K, C1), axis=2)
    o = jnp.maximum(ftsT + zm, 0.0)
    out_ref[...] = jnp.transpose(o, (0, 2, 1)) * mask


def _ec2_stats(fts1_ref, feat_ref, wsc_ref, w_ref, gb_ref, acc_ref, idx_s):
    s = pl.program_id(0)
    blk = pl.program_id(1)

    @pl.when(jnp.logical_and(s == 0, blk == 0))
    def _zero():
        acc_ref[...] = jnp.zeros_like(acc_ref)

    f1 = fts1_ref[...]
    mask = _mask_of(feat_ref[...])
    f1T = jnp.transpose(f1, (0, 2, 1)).reshape(BB * N, C1)

    @pl.when(s == 0)
    def _s0():
        pts = f1 + (1.0 - mask) * 1e9
        idx_s[:, pl.ds(blk * BB * N, BB * N)] = _knn_idx(pts).T
        _acc(acc_ref, 0, _mm(f1T, wsc_ref[...]))

    @pl.when(s > 0)
    def _rest():
        ftsT = f1T.reshape(BB, N, C1)
        idx8 = idx_s[:, pl.ds(blk * BB * N, BB * N)].T
        y = _mm(_gather_edges(ftsT, idx8[:, 1:]), w_ref[0:64, :])

        @pl.when(s == 1)
        def _():
            _acc(acc_ref, 2, y)

        @pl.when(s > 1)
        def _d1():
            sc, sh = _bn_params(acc_ref[2, :], acc_ref[3, :], CNTK,
                                gb_ref[0, :], gb_ref[1, :])
            y2 = _mm(jnp.maximum(y * sc + sh, 0.0), w_ref[64:128, :])

            @pl.when(s == 2)
            def _():
                _acc(acc_ref, 4, y2)

            @pl.when(s == 3)
            def _d2():
                sc2, sh2 = _bn_params(acc_ref[4, :], acc_ref[5, :], CNTK,
                                      gb_ref[2, :], gb_ref[3, :])
                y3 = _mm(jnp.maximum(y2 * sc2 + sh2, 0.0), w_ref[128:192, :])
                _acc(acc_ref, 6, y3)


def _ec2_apply(fts1_ref, feat_ref, wsc_ref, w_ref, gb_ref, acc_ref, out_ref):
    f1 = fts1_ref[...]
    mask = _mask_of(feat_ref[...])
    f1T = jnp.transpose(f1, (0, 2, 1)).reshape(BB * N, C1)
    pts = f1 + (1.0 - mask) * 1e9
    idx8 = _knn_idx(pts)
    ftsT = f1T.reshape(BB, N, C1)
    y = _mm(_gather_edges(ftsT, idx8[:, 1:]), w_ref[0:64, :])
    sc, sh = _bn_params(acc_ref[2, :], acc_ref[3, :], CNTK,
                        gb_ref[0, :], gb_ref[1, :])
    y2 = _mm(jnp.maximum(y * sc + sh, 0.0), w_ref[64:128, :])
    sc2, sh2 = _bn_params(acc_ref[4, :], acc_ref[5, :], CNTK,
                          gb_ref[2, :], gb_ref[3, :])
    y3 = _mm(jnp.maximum(y2 * sc2 + sh2, 0.0), w_ref[128:192, :])
    sc3, sh3 = _bn_params(acc_ref[6, :], acc_ref[7, :], CNTK,
                          gb_ref[4, :], gb_ref[5, :])
    z3 = jnp.maximum(y3 * sc3 + sh3, 0.0)
    zm = jnp.mean(z3.reshape(BB, N, K, C2), axis=2)
    scp, shp = _bn_params(acc_ref[0, :], acc_ref[1, :], CNTF,
                          gb_ref[6, :], gb_ref[7, :])
    scv = _mm(f1T, wsc_ref[...]).reshape(BB, N, C2) * scp + shp
    o = jnp.maximum(scv + zm, 0.0)
    out_ref[...] = jnp.transpose(o, (0, 2, 1)) * mask


def _yf(fts1_ref, fts2_ref, fusw_ref):
    cat = jnp.concatenate(
        [jnp.transpose(fts1_ref[...], (0, 2, 1)),
         jnp.transpose(fts2_ref[...], (0, 2, 1))], axis=2)
    return _mm(cat.reshape(BB * N, 96), fusw_ref[...])


def _head_stats(fts1_ref, fts2_ref, fusw_ref, acc_ref):
    blk = pl.program_id(0)

    @pl.when(blk == 0)
    def _zero():
        acc_ref[...] = jnp.zeros_like(acc_ref)

    _acc(acc_ref, 0, _yf(fts1_ref, fts2_ref, fusw_ref))


def _head_apply(fts1_ref, fts2_ref, feat_ref, fusw_ref, gb_ref, fc1w_ref,
                fc2w_ref, acc_ref, out_ref):
    mask = _mask_of(feat_ref[...])
    yf = _yf(fts1_ref, fts2_ref, fusw_ref)
    sc, sh = _bn_params(acc_ref[0, :], acc_ref[1, :], CNTF,
                        gb_ref[0, :], gb_ref[1, :])
    h = jnp.maximum(yf * sc + sh, 0.0).reshape(BB, N, 128)
    h = h * jnp.transpose(mask, (0, 2, 1))
    counts = jnp.maximum(jnp.sum(mask, axis=2), 1.0)
    pooled = jnp.sum(h, axis=1) / counts
    a = jnp.maximum(_mm(pooled, fc1w_ref[...]) + gb_ref[2, :], 0.0)
    out_ref[...] = _mm(a, fc2w_ref[...]) + gb_ref[3, 0:16]


def _bcast2(shape):
    return pl.BlockSpec(shape, lambda s, b: (0,) * len(shape))


def _bblk2(ch):
    return pl.BlockSpec((BB, ch, N), lambda s, b: (b, 0, 0))


def _bcast1(shape):
    return pl.BlockSpec(shape, lambda b: (0,) * len(shape))


def _bblk1(ch):
    return pl.BlockSpec((BB, ch, N), lambda b: (b, 0, 0))


def kernel(points, features, bn_fts_g, bn_fts_b, ec1_W0, ec1_W1, ec1_W2,
           ec1_g0, ec1_b0, ec1_g1, ec1_b1, ec1_g2, ec1_b2,
           ec2_W0, ec2_W1, ec2_W2, ec2_g0, ec2_b0, ec2_g1, ec2_b1,
           ec2_g2, ec2_b2, ec2_Wsc, ec2_sc_g, ec2_sc_b,
           fus_W, fus_g, fus_b, fc1_W, fc1_b, fc2_W, fc2_b):
    f32 = jnp.float32
    pts8 = jnp.concatenate(
        [points, jnp.zeros((B, 5, N), f32)], axis=1)

    ec1_w = jnp.zeros((96, 64), f32)
    ec1_w = ec1_w.at[0:32, 0:64].set(ec1_W0)
    ec1_w = ec1_w.at[32:64, 0:32].set(ec1_W1)
    ec1_w = ec1_w.at[64:96, 0:32].set(ec1_W2)
    ec1_gb = jnp.stack([ec1_g0, ec1_b0, ec1_g1, ec1_b1, ec1_g2, ec1_b2])
    g0 = bn_fts_g.reshape(1, C1)
    b0 = bn_fts_b.reshape(1, C1)

    ec1_in = [pts8, features, g0, b0, ec1_w, ec1_gb]
    acc1 = pl.pallas_call(
        _ec1_stats,
        grid=(4, NB),
        in_specs=[_bblk2(8), _bblk2(C1), _bcast2((1, C1)), _bcast2((1, C1)),
                  _bcast2((96, 64)), _bcast2((6, C1))],
        out_specs=_bcast2((8, C1)),
        out_shape=jax.ShapeDtypeStruct((8, C1), f32),
        scratch_shapes=[pltpu.VMEM((8, B * N), jnp.int32)],
    )(*ec1_in)

    fts1 = pl.pallas_call(
        _ec1_apply,
        grid=(NB,),
        in_specs=[_bblk1(8), _bblk1(C1), _bcast1((1, C1)), _bcast1((1, C1)),
                  _bcast1((96, 64)), _bcast1((6, C1)), _bcast1((8, C1))],
        out_specs=_bblk1(C1),
        out_shape=jax.ShapeDtypeStruct((B, C1, N), f32),
    )(*ec1_in, acc1)

    ec2_w = jnp.concatenate([ec2_W0, ec2_W1, ec2_W2], axis=0)
    ec2_gb = jnp.stack([ec2_g0, ec2_b0, ec2_g1, ec2_b1, ec2_g2, ec2_b2,
                        ec2_sc_g, ec2_sc_b])

    ec2_in = [fts1, features, ec2_Wsc, ec2_w, ec2_gb]
    acc2 = pl.pallas_call(
        _ec2_stats,
        grid=(4, NB),
        in_specs=[_bblk2(C1), _bblk2(C1), _bcast2((C2, C1)),
                  _bcast2((192, C2)), _bcast2((8, C2))],
        out_specs=_bcast2((8, C2)),
        out_shape=jax.ShapeDtypeStruct((8, C2), f32),
        scratch_shapes=[pltpu.VMEM((8, B * N), jnp.int32)],
    )(*ec2_in)

    fts2 = pl.pallas_call(
        _ec2_apply,
        grid=(NB,),
        in_specs=[_bblk1(C1), _bblk1(C1), _bcast1((C2, C1)),
                  _bcast1((192, C2)), _bcast1((8, C2)), _bcast1((8, C2))],
        out_specs=_bblk1(C2),
        out_shape=jax.ShapeDtypeStruct((B, C2, N), f32),
    )(*ec2_in, acc2)

    acch = pl.pallas_call(
        _head_stats,
        grid=(NB,),
        in_specs=[_bblk1(C1), _bblk1(C2), _bcast1((128, 96))],
        out_specs=_bcast1((2, 128)),
        out_shape=jax.ShapeDtypeStruct((2, 128), f32),
    )(fts1, fts2, fus_W)

    fc2_pad = jnp.zeros((16, 128), f32).at[0:10, :].set(fc2_W)
    head_gb = jnp.stack([fus_g, fus_b, fc1_b,
                         jnp.zeros((128,), f32).at[0:10].set(fc2_b)])

    out16 = pl.pallas_call(
        _head_apply,
        grid=(NB,),
        in_specs=[_bblk1(C1), _bblk1(C2), _bblk1(C1), _bcast1((128, 96)),
                  _bcast1((4, 128)), _bcast1((128, 128)), _bcast1((16, 128)),
                  _bcast1((2, 128))],
        out_specs=pl.BlockSpec((BB, 16), lambda b: (b, 0)),
        out_shape=jax.ShapeDtypeStruct((B, 16), f32),
    )(fts1, fts2, features, fus_W, head_gb, fc1_W, fc2_pad, acch)

    return out16[:, 0:10]


# speedup vs baseline: 5.1040x; 1.6905x over previous
import jax
import jax.numpy as jnp
from jax.experimental import pallas as pl
from jax.experimental.pallas import tpu as pltpu

B = 256
N = 128
K = 7
C1 = 32
C2 = 64
BB = 8
NB = B // BB
NK = N * K
EPS = 1e-5
NEG = -3.0e38
CNTF = float(B * N)
CNTK = float(B * N * K)


def _bn_params(s, ss, cnt, g, b):
    m = s / cnt
    v = ss / cnt - m * m
    scale = g * jax.lax.rsqrt(v + EPS)
    shift = b - m * scale
    return scale, shift


def _mask_of(f):
    return (jnp.sum(jnp.abs(f), axis=1, keepdims=True) != 0.0).astype(jnp.float32)


def _knn_idx(pts):
    rows = []
    for i in range(BB):
        p = pts[i]
        pt = p.T
        g = jax.lax.dot_general(pt, p, (((1,), (0,)), ((), ())),
                                preferred_element_type=jnp.float32)
        xx = jnp.sum(pt * pt, axis=1, keepdims=True)
        rows.append(2.0 * g - xx - xx.T)
    pd = jnp.concatenate(rows, axis=0)
    lane = jax.lax.broadcasted_iota(jnp.int32, (BB * N, N), 1)
    cols = []
    for _ in range(K + 1):
        m = jnp.max(pd, axis=1, keepdims=True)
        cand = jnp.where(pd >= m, lane, N)
        j = jnp.min(cand, axis=1, keepdims=True)
        cols.append(j)
        pd = jnp.where(lane == j, NEG, pd)
    return jnp.concatenate(cols, axis=1)


def _gather_edges(ftsT, idx7):
    lane = jax.lax.broadcasted_iota(jnp.int32, (N, N), 1)
    outs = []
    for i in range(BB):
        idxi = idx7[i * N:(i + 1) * N, :]
        oh = jnp.concatenate(
            [(idxi[:, j:j + 1] == lane).astype(jnp.float32) for j in range(K)],
            axis=0)
        nb = jax.lax.dot_general(oh, ftsT[i], (((1,), (0,)), ((), ())),
                                 preferred_element_type=jnp.float32)
        ctr = jnp.concatenate([ftsT[i]] * K, axis=0)
        outs.append(jnp.concatenate([ctr, nb - ctr], axis=1))
    return jnp.concatenate(outs, axis=0)


def _mm(x, w):
    return jax.lax.dot_general(x, w, (((1,), (1,)), ((), ())),
                               preferred_element_type=jnp.float32)


def _acc(ref, row, y):
    ref[row, :] += jnp.sum(y, axis=0)
    ref[row + 1, :] += jnp.sum(y * y, axis=0)


def _ec1_fn(f, mask, acc, g0, b0):
    sc0, sh0 = _bn_params(acc[0, :], acc[1, :], CNTF, g0[0, :], b0[0, :])
    return (f * sc0[None, :, None] + sh0[None, :, None]) * mask


def _ec1_stats(points_ref, feat_ref, g0_ref, b0_ref, w_ref, gb_ref,
               acc_ref, idx_s):
    s = pl.program_id(0)
    blk = pl.program_id(1)

    @pl.when(jnp.logical_and(s == 0, blk == 0))
    def _zero():
        acc_ref[...] = jnp.zeros_like(acc_ref)

    f = feat_ref[...]
    mask = _mask_of(f)

    @pl.when(s == 0)
    def _s0():
        pts = points_ref[...] * mask + (1.0 - mask) * 1e9
        idx_s[:, pl.ds(blk * BB * N, BB * N)] = _knn_idx(pts).T
        acc_ref[0, :] += jnp.sum(f, axis=(0, 2))
        acc_ref[1, :] += jnp.sum(f * f, axis=(0, 2))

    @pl.when(s > 0)
    def _rest():
        fn = _ec1_fn(f, mask, acc_ref, g0_ref, b0_ref)
        ftsT = jnp.transpose(fn, (0, 2, 1))
        idx8 = idx_s[:, pl.ds(blk * BB * N, BB * N)].T
        y = _mm(_gather_edges(ftsT, idx8[:, 1:]), w_ref[0:32, 0:64])

        @pl.when(s == 1)
        def _():
            _acc(acc_ref, 2, y)

        @pl.when(s > 1)
        def _d1():
            sc, sh = _bn_params(acc_ref[2, :], acc_ref[3, :], CNTK,
                                gb_ref[0, :], gb_ref[1, :])
            y2 = _mm(jnp.maximum(y * sc + sh, 0.0), w_ref[32:64, 0:32])

            @pl.when(s == 2)
            def _():
                _acc(acc_ref, 4, y2)

            @pl.when(s == 3)
            def _d2():
                sc2, sh2 = _bn_params(acc_ref[4, :], acc_ref[5, :], CNTK,
                                      gb_ref[2, :], gb_ref[3, :])
                y3 = _mm(jnp.maximum(y2 * sc2 + sh2, 0.0), w_ref[64:96, 0:32])
                _acc(acc_ref, 6, y3)


def _ec1_apply(points_ref, feat_ref, g0_ref, b0_ref, w_ref, gb_ref, acc_ref,
               out_ref):
    f = feat_ref[...]
    mask = _mask_of(f)
    pts = points_ref[...] * mask + (1.0 - mask) * 1e9
    idx8 = _knn_idx(pts)
    fn = _ec1_fn(f, mask, acc_ref, g0_ref, b0_ref)
    ftsT = jnp.transpose(fn, (0, 2, 1))
    y = _mm(_gather_edges(ftsT, idx8[:, 1:]), w_ref[0:32, 0:64])
    sc, sh = _bn_params(acc_ref[2, :], acc_ref[3, :], CNTK,
                        gb_ref[0, :], gb_ref[1, :])
    y2 = _mm(jnp.maximum(y * sc + sh, 0.0), w_ref[32:64, 0:32])
    sc2, sh2 = _bn_params(acc_ref[4, :], acc_ref[5, :], CNTK,
                          gb_ref[2, :], gb_ref[3, :])
    y3 = _mm(jnp.maximum(y2 * sc2 + sh2, 0.0), w_ref[64:96, 0:32])
    sc3, sh3 = _bn_params(acc_ref[6, :], acc_ref[7, :], CNTK,
                          gb_ref[4, :], gb_ref[5, :])
    z3 = jnp.maximum(y3 * sc3 + sh3, 0.0)
    zm = jnp.mean(z3.reshape(BB, K, N, C1), axis=1)
    o = jnp.maximum(ftsT + zm, 0.0)
    out_ref[...] = jnp.transpose(o, (0, 2, 1)) * mask


def _ec2_stats(fts1_ref, feat_ref, wsc_ref, w_ref, gb_ref, acc_ref, idx_s):
    s = pl.program_id(0)
    blk = pl.program_id(1)

    @pl.when(jnp.logical_and(s == 0, blk == 0))
    def _zero():
        acc_ref[...] = jnp.zeros_like(acc_ref)

    f1 = fts1_ref[...]
    mask = _mask_of(feat_ref[...])
    f1T = jnp.transpose(f1, (0, 2, 1)).reshape(BB * N, C1)

    @pl.when(s == 0)
    def _s0():
        pts = f1 + (1.0 - mask) * 1e9
        idx_s[:, pl.ds(blk * BB * N, BB * N)] = _knn_idx(pts).T
        _acc(acc_ref, 0, _mm(f1T, wsc_ref[...]))

    @pl.when(s > 0)
    def _rest():
        ftsT = f1T.reshape(BB, N, C1)
        idx8 = idx_s[:, pl.ds(blk * BB * N, BB * N)].T
        y = _mm(_gather_edges(ftsT, idx8[:, 1:]), w_ref[0:64, :])

        @pl.when(s == 1)
        def _():
            _acc(acc_ref, 2, y)

        @pl.when(s > 1)
        def _d1():
            sc, sh = _bn_params(acc_ref[2, :], acc_ref[3, :], CNTK,
                                gb_ref[0, :], gb_ref[1, :])
            y2 = _mm(jnp.maximum(y * sc + sh, 0.0), w_ref[64:128, :])

            @pl.when(s == 2)
            def _():
                _acc(acc_ref, 4, y2)

            @pl.when(s == 3)
            def _d2():
                sc2, sh2 = _bn_params(acc_ref[4, :], acc_ref[5, :], CNTK,
                                      gb_ref[2, :], gb_ref[3, :])
                y3 = _mm(jnp.maximum(y2 * sc2 + sh2, 0.0), w_ref[128:192, :])
                _acc(acc_ref, 6, y3)


def _ec2_apply(fts1_ref, feat_ref, wsc_ref, w_ref, gb_ref, acc_ref, out_ref):
    f1 = fts1_ref[...]
    mask = _mask_of(feat_ref[...])
    f1T = jnp.transpose(f1, (0, 2, 1)).reshape(BB * N, C1)
    pts = f1 + (1.0 - mask) * 1e9
    idx8 = _knn_idx(pts)
    ftsT = f1T.reshape(BB, N, C1)
    y = _mm(_gather_edges(ftsT, idx8[:, 1:]), w_ref[0:64, :])
    sc, sh = _bn_params(acc_ref[2, :], acc_ref[3, :], CNTK,
                        gb_ref[0, :], gb_ref[1, :])
    y2 = _mm(jnp.maximum(y * sc + sh, 0.0), w_ref[64:128, :])
    sc2, sh2 = _bn_params(acc_ref[4, :], acc_ref[5, :], CNTK,
                          gb_ref[2, :], gb_ref[3, :])
    y3 = _mm(jnp.maximum(y2 * sc2 + sh2, 0.0), w_ref[128:192, :])
    sc3, sh3 = _bn_params(acc_ref[6, :], acc_ref[7, :], CNTK,
                          gb_ref[4, :], gb_ref[5, :])
    z3 = jnp.maximum(y3 * sc3 + sh3, 0.0)
    zm = jnp.mean(z3.reshape(BB, K, N, C2), axis=1)
    scp, shp = _bn_params(acc_ref[0, :], acc_ref[1, :], CNTF,
                          gb_ref[6, :], gb_ref[7, :])
    scv = _mm(f1T, wsc_ref[...]).reshape(BB, N, C2) * scp + shp
    o = jnp.maximum(scv + zm, 0.0)
    out_ref[...] = jnp.transpose(o, (0, 2, 1)) * mask


def _yf(fts1_ref, fts2_ref, fusw_ref):
    cat = jnp.concatenate(
        [jnp.transpose(fts1_ref[...], (0, 2, 1)),
         jnp.transpose(fts2_ref[...], (0, 2, 1))], axis=2)
    return _mm(cat.reshape(BB * N, 96), fusw_ref[...])


def _head_stats(fts1_ref, fts2_ref, fusw_ref, acc_ref):
    blk = pl.program_id(0)

    @pl.when(blk == 0)
    def _zero():
        acc_ref[...] = jnp.zeros_like(acc_ref)

    _acc(acc_ref, 0, _yf(fts1_ref, fts2_ref, fusw_ref))


def _head_apply(fts1_ref, fts2_ref, feat_ref, fusw_ref, gb_ref, fc1w_ref,
                fc2w_ref, acc_ref, out_ref):
    mask = _mask_of(feat_ref[...])
    yf = _yf(fts1_ref, fts2_ref, fusw_ref)
    sc, sh = _bn_params(acc_ref[0, :], acc_ref[1, :], CNTF,
                        gb_ref[0, :], gb_ref[1, :])
    h = jnp.maximum(yf * sc + sh, 0.0).reshape(BB, N, 128)
    h = h * jnp.transpose(mask, (0, 2, 1))
    counts = jnp.maximum(jnp.sum(mask, axis=2), 1.0)
    pooled = jnp.sum(h, axis=1) / counts
    a = jnp.maximum(_mm(pooled, fc1w_ref[...]) + gb_ref[2, :], 0.0)
    out_ref[...] = _mm(a, fc2w_ref[...]) + gb_ref[3, 0:16]


def _bcast2(shape):
    return pl.BlockSpec(shape, lambda s, b: (0,) * len(shape))


def _bblk2(ch):
    return pl.BlockSpec((BB, ch, N), lambda s, b: (b, 0, 0))


def _bcast1(shape):
    return pl.BlockSpec(shape, lambda b: (0,) * len(shape))


def _bblk1(ch):
    return pl.BlockSpec((BB, ch, N), lambda b: (b, 0, 0))


def kernel(points, features, bn_fts_g, bn_fts_b, ec1_W0, ec1_W1, ec1_W2,
           ec1_g0, ec1_b0, ec1_g1, ec1_b1, ec1_g2, ec1_b2,
           ec2_W0, ec2_W1, ec2_W2, ec2_g0, ec2_b0, ec2_g1, ec2_b1,
           ec2_g2, ec2_b2, ec2_Wsc, ec2_sc_g, ec2_sc_b,
           fus_W, fus_g, fus_b, fc1_W, fc1_b, fc2_W, fc2_b):
    f32 = jnp.float32
    pts8 = jnp.concatenate(
        [points, jnp.zeros((B, 5, N), f32)], axis=1)

    ec1_w = jnp.zeros((96, 64), f32)
    ec1_w = ec1_w.at[0:32, 0:64].set(ec1_W0)
    ec1_w = ec1_w.at[32:64, 0:32].set(ec1_W1)
    ec1_w = ec1_w.at[64:96, 0:32].set(ec1_W2)
    ec1_gb = jnp.stack([ec1_g0, ec1_b0, ec1_g1, ec1_b1, ec1_g2, ec1_b2])
    g0 = bn_fts_g.reshape(1, C1)
    b0 = bn_fts_b.reshape(1, C1)

    ec1_in = [pts8, features, g0, b0, ec1_w, ec1_gb]
    acc1 = pl.pallas_call(
        _ec1_stats,
        grid=(4, NB),
        in_specs=[_bblk2(8), _bblk2(C1), _bcast2((1, C1)), _bcast2((1, C1)),
                  _bcast2((96, 64)), _bcast2((6, C1))],
        out_specs=_bcast2((8, C1)),
        out_shape=jax.ShapeDtypeStruct((8, C1), f32),
        scratch_shapes=[pltpu.VMEM((8, B * N), jnp.int32)],
    )(*ec1_in)

    fts1 = pl.pallas_call(
        _ec1_apply,
        grid=(NB,),
        in_specs=[_bblk1(8), _bblk1(C1), _bcast1((1, C1)), _bcast1((1, C1)),
                  _bcast1((96, 64)), _bcast1((6, C1)), _bcast1((8, C1))],
        out_specs=_bblk1(C1),
        out_shape=jax.ShapeDtypeStruct((B, C1, N), f32),
    )(*ec1_in, acc1)

    ec2_w = jnp.concatenate([ec2_W0, ec2_W1, ec2_W2], axis=0)
    ec2_gb = jnp.stack([ec2_g0, ec2_b0, ec2_g1, ec2_b1, ec2_g2, ec2_b2,
                        ec2_sc_g, ec2_sc_b])

    ec2_in = [fts1, features, ec2_Wsc, ec2_w, ec2_gb]
    acc2 = pl.pallas_call(
        _ec2_stats,
        grid=(4, NB),
        in_specs=[_bblk2(C1), _bblk2(C1), _bcast2((C2, C1)),
                  _bcast2((192, C2)), _bcast2((8, C2))],
        out_specs=_bcast2((8, C2)),
        out_shape=jax.ShapeDtypeStruct((8, C2), f32),
        scratch_shapes=[pltpu.VMEM((8, B * N), jnp.int32)],
    )(*ec2_in)

    fts2 = pl.pallas_call(
        _ec2_apply,
        grid=(NB,),
        in_specs=[_bblk1(C1), _bblk1(C1), _bcast1((C2, C1)),
                  _bcast1((192, C2)), _bcast1((8, C2)), _bcast1((8, C2))],
        out_specs=_bblk1(C2),
        out_shape=jax.ShapeDtypeStruct((B, C2, N), f32),
    )(*ec2_in, acc2)

    acch = pl.pallas_call(
        _head_stats,
        grid=(NB,),
        in_specs=[_bblk1(C1), _bblk1(C2), _bcast1((128, 96))],
        out_specs=_bcast1((2, 128)),
        out_shape=jax.ShapeDtypeStruct((2, 128), f32),
    )(fts1, fts2, fus_W)

    fc2_pad = jnp.zeros((16, 128), f32).at[0:10, :].set(fc2_W)
    head_gb = jnp.stack([fus_g, fus_b, fc1_b,
                         jnp.zeros((128,), f32).at[0:10].set(fc2_b)])

    out16 = pl.pallas_call(
        _head_apply,
        grid=(NB,),
        in_specs=[_bblk1(C1), _bblk1(C2), _bblk1(C1), _bcast1((128, 96)),
                  _bcast1((4, 128)), _bcast1((128, 128)), _bcast1((16, 128)),
                  _bcast1((2, 128))],
        out_specs=pl.BlockSpec((BB, 16), lambda b: (b, 0)),
        out_shape=jax.ShapeDtypeStruct((B, 16), f32),
    )(fts1, fts2, features, fus_W, head_gb, fc1_W, fc2_pad, acch)

    return out16[:, 0:10]


# speedup vs baseline: 6.4357x; 1.2609x over previous
import jax
import jax.numpy as jnp
from jax.experimental import pallas as pl
from jax.experimental.pallas import tpu as pltpu

B = 256
N = 128
K = 7
C1 = 32
C2 = 64
BB = 8
NB = B // BB
NK = N * K
EPS = 1e-5
NEG = -3.0e38
CNTF = float(B * N)
CNTK = float(B * N * K)


def _bn_params(s, ss, cnt, g, b):
    m = s / cnt
    v = ss / cnt - m * m
    scale = g * jax.lax.rsqrt(v + EPS)
    shift = b - m * scale
    return scale, shift


def _mask_of(f):
    return (jnp.sum(jnp.abs(f), axis=1, keepdims=True) != 0.0).astype(jnp.float32)


def _knn_idx(pts):
    dg = jax.lax.broadcasted_iota(jnp.int32, (N, N), 0) == \
        jax.lax.broadcasted_iota(jnp.int32, (N, N), 1)
    rows = []
    for i in range(BB):
        p = pts[i]
        pt = p.T
        g = jax.lax.dot_general(pt, p, (((1,), (0,)), ((), ())),
                                preferred_element_type=jnp.float32)
        xx = jnp.sum(pt * pt, axis=1, keepdims=True)
        rows.append(jnp.where(dg, NEG, 2.0 * g - xx - xx.T))
    pd = jnp.concatenate(rows, axis=0)
    lane = jax.lax.broadcasted_iota(jnp.int32, (BB * N, N), 1)
    cols = []
    for _ in range(K):
        m = jnp.max(pd, axis=1, keepdims=True)
        sel = pd >= m
        j = jnp.min(jnp.where(sel, lane, N), axis=1, keepdims=True)
        cols.append(j)
        pd = jnp.where(sel, NEG, pd)
    return jnp.concatenate(cols, axis=1)


def _mm(x, w):
    return jax.lax.dot_general(x, w, (((1,), (1,)), ((), ())),
                               preferred_element_type=jnp.float32)


def _edge_conv1(ftsT, idx7, w):
    C = ftsT.shape[-1]
    wd = w[:, 0:C] - w[:, C:2 * C]
    wb = w[:, C:2 * C]
    lane = jax.lax.broadcasted_iota(jnp.int32, (N, N), 1)
    outs = []
    for i in range(BB):
        base = _mm(ftsT[i], wd)
        gsrc = _mm(ftsT[i], wb)
        idxi = idx7[i * N:(i + 1) * N, :]
        oh = jnp.concatenate(
            [(idxi[:, j:j + 1] == lane).astype(jnp.float32) for j in range(K)],
            axis=0)
        nb = jax.lax.dot_general(oh, gsrc, (((1,), (0,)), ((), ())),
                                 preferred_element_type=jnp.float32)
        outs.append(nb + jnp.concatenate([base] * K, axis=0))
    return jnp.concatenate(outs, axis=0)


def _acc(ref, row, y):
    ones = jnp.ones((1, y.shape[0]), jnp.float32)
    s = jax.lax.dot_general(ones, y, (((1,), (0,)), ((), ())),
                            preferred_element_type=jnp.float32)
    g = jax.lax.dot_general(y, y, (((0,), (0,)), ((), ())),
                            preferred_element_type=jnp.float32)
    c = y.shape[1]
    eye = (jax.lax.broadcasted_iota(jnp.int32, (c, c), 0) ==
           jax.lax.broadcasted_iota(jnp.int32, (c, c), 1))
    ss = jnp.sum(jnp.where(eye, g, 0.0), axis=0)
    ref[row, :] += s[0, :]
    ref[row + 1, :] += ss


def _ec1_fn(f, mask, acc, g0, b0):
    sc0, sh0 = _bn_params(acc[0, :], acc[1, :], CNTF, g0[0, :], b0[0, :])
    return (f * sc0[None, :, None] + sh0[None, :, None]) * mask


def _ec1_stats(points_ref, feat_ref, g0_ref, b0_ref, w_ref, gb_ref,
               acc_ref, idx_s):
    s = pl.program_id(0)
    blk = pl.program_id(1)

    @pl.when(jnp.logical_and(s == 0, blk == 0))
    def _zero():
        acc_ref[...] = jnp.zeros_like(acc_ref)

    f = feat_ref[...]
    mask = _mask_of(f)

    @pl.when(s == 0)
    def _s0():
        pts = points_ref[...] * mask + (1.0 - mask) * 1e9
        idx = _knn_idx(pts)
        idx_s[:, pl.ds(blk * BB * N, BB * N)] = jnp.concatenate(
            [idx, idx[:, 6:7]], axis=1).T
        acc_ref[0, :] += jnp.sum(f, axis=(0, 2))
        acc_ref[1, :] += jnp.sum(f * f, axis=(0, 2))

    @pl.when(s > 0)
    def _rest():
        fn = _ec1_fn(f, mask, acc_ref, g0_ref, b0_ref)
        ftsT = jnp.transpose(fn, (0, 2, 1))
        idx7 = idx_s[:, pl.ds(blk * BB * N, BB * N)].T[:, 0:7]
        y = _edge_conv1(ftsT, idx7, w_ref[0:32, 0:64])

        @pl.when(s == 1)
        def _():
            _acc(acc_ref, 2, y)

        @pl.when(s > 1)
        def _d1():
            sc, sh = _bn_params(acc_ref[2, :], acc_ref[3, :], CNTK,
                                gb_ref[0, :], gb_ref[1, :])
            y2 = _mm(jnp.maximum(y * sc + sh, 0.0), w_ref[32:64, 0:32])

            @pl.when(s == 2)
            def _():
                _acc(acc_ref, 4, y2)

            @pl.when(s == 3)
            def _d2():
                sc2, sh2 = _bn_params(acc_ref[4, :], acc_ref[5, :], CNTK,
                                      gb_ref[2, :], gb_ref[3, :])
                y3 = _mm(jnp.maximum(y2 * sc2 + sh2, 0.0), w_ref[64:96, 0:32])
                _acc(acc_ref, 6, y3)


def _ec1_apply(points_ref, feat_ref, g0_ref, b0_ref, w_ref, gb_ref, acc_ref,
               out_ref):
    f = feat_ref[...]
    mask = _mask_of(f)
    pts = points_ref[...] * mask + (1.0 - mask) * 1e9
    idx7 = _knn_idx(pts)
    fn = _ec1_fn(f, mask, acc_ref, g0_ref, b0_ref)
    ftsT = jnp.transpose(fn, (0, 2, 1))
    y = _edge_conv1(ftsT, idx7, w_ref[0:32, 0:64])
    sc, sh = _bn_params(acc_ref[2, :], acc_ref[3, :], CNTK,
                        gb_ref[0, :], gb_ref[1, :])
    y2 = _mm(jnp.maximum(y * sc + sh, 0.0), w_ref[32:64, 0:32])
    sc2, sh2 = _bn_params(acc_ref[4, :], acc_ref[5, :], CNTK,
                          gb_ref[2, :], gb_ref[3, :])
    y3 = _mm(jnp.maximum(y2 * sc2 + sh2, 0.0), w_ref[64:96, 0:32])
    sc3, sh3 = _bn_params(acc_ref[6, :], acc_ref[7, :], CNTK,
                          gb_ref[4, :], gb_ref[5, :])
    z3 = jnp.maximum(y3 * sc3 + sh3, 0.0)
    zm = jnp.mean(z3.reshape(BB, K, N, C1), axis=1)
    o = jnp.maximum(ftsT + zm, 0.0)
    out_ref[...] = jnp.transpose(o, (0, 2, 1)) * mask


def _ec2_stats(fts1_ref, feat_ref, wsc_ref, w_ref, gb_ref, acc_ref, idx_s):
    s = pl.program_id(0)
    blk = pl.program_id(1)

    @pl.when(jnp.logical_and(s == 0, blk == 0))
    def _zero():
        acc_ref[...] = jnp.zeros_like(acc_ref)

    f1 = fts1_ref[...]
    mask = _mask_of(feat_ref[...])
    f1T = jnp.transpose(f1, (0, 2, 1)).reshape(BB * N, C1)

    @pl.when(s == 0)
    def _s0():
        pts = f1 + (1.0 - mask) * 1e9
        idx = _knn_idx(pts)
        idx_s[:, pl.ds(blk * BB * N, BB * N)] = jnp.concatenate(
            [idx, idx[:, 6:7]], axis=1).T
        _acc(acc_ref, 0, _mm(f1T, wsc_ref[...]))

    @pl.when(s > 0)
    def _rest():
        ftsT = f1T.reshape(BB, N, C1)
        idx7 = idx_s[:, pl.ds(blk * BB * N, BB * N)].T[:, 0:7]
        y = _edge_conv1(ftsT, idx7, w_ref[0:64, :])

        @pl.when(s == 1)
        def _():
            _acc(acc_ref, 2, y)

        @pl.when(s > 1)
        def _d1():
            sc, sh = _bn_params(acc_ref[2, :], acc_ref[3, :], CNTK,
                                gb_ref[0, :], gb_ref[1, :])
            y2 = _mm(jnp.maximum(y * sc + sh, 0.0), w_ref[64:128, :])

            @pl.when(s == 2)
            def _():
                _acc(acc_ref, 4, y2)

            @pl.when(s == 3)
            def _d2():
                sc2, sh2 = _bn_params(acc_ref[4, :], acc_ref[5, :], CNTK,
                                      gb_ref[2, :], gb_ref[3, :])
                y3 = _mm(jnp.maximum(y2 * sc2 + sh2, 0.0), w_ref[128:192, :])
                _acc(acc_ref, 6, y3)


def _ec2_apply(fts1_ref, feat_ref, wsc_ref, w_ref, gb_ref, acc_ref, out_ref):
    f1 = fts1_ref[...]
    mask = _mask_of(feat_ref[...])
    f1T = jnp.transpose(f1, (0, 2, 1)).reshape(BB * N, C1)
    pts = f1 + (1.0 - mask) * 1e9
    idx7 = _knn_idx(pts)
    ftsT = f1T.reshape(BB, N, C1)
    y = _edge_conv1(ftsT, idx7, w_ref[0:64, :])
    sc, sh = _bn_params(acc_ref[2, :], acc_ref[3, :], CNTK,
                        gb_ref[0, :], gb_ref[1, :])
    y2 = _mm(jnp.maximum(y * sc + sh, 0.0), w_ref[64:128, :])
    sc2, sh2 = _bn_params(acc_ref[4, :], acc_ref[5, :], CNTK,
                          gb_ref[2, :], gb_ref[3, :])
    y3 = _mm(jnp.maximum(y2 * sc2 + sh2, 0.0), w_ref[128:192, :])
    sc3, sh3 = _bn_params(acc_ref[6, :], acc_ref[7, :], CNTK,
                          gb_ref[4, :], gb_ref[5, :])
    z3 = jnp.maximum(y3 * sc3 + sh3, 0.0)
    zm = jnp.mean(z3.reshape(BB, K, N, C2), axis=1)
    scp, shp = _bn_params(acc_ref[0, :], acc_ref[1, :], CNTF,
                          gb_ref[6, :], gb_ref[7, :])
    scv = _mm(f1T, wsc_ref[...]).reshape(BB, N, C2) * scp + shp
    o = jnp.maximum(scv + zm, 0.0)
    out_ref[...] = jnp.transpose(o, (0, 2, 1)) * mask


def _yf(fts1_ref, fts2_ref, fusw_ref):
    cat = jnp.concatenate(
        [jnp.transpose(fts1_ref[...], (0, 2, 1)),
         jnp.transpose(fts2_ref[...], (0, 2, 1))], axis=2)
    return _mm(cat.reshape(BB * N, 96), fusw_ref[...])


def _head_stats(fts1_ref, fts2_ref, fusw_ref, acc_ref):
    blk = pl.program_id(0)

    @pl.when(blk == 0)
    def _zero():
        acc_ref[...] = jnp.zeros_like(acc_ref)

    _acc(acc_ref, 0, _yf(fts1_ref, fts2_ref, fusw_ref))


def _head_apply(fts1_ref, fts2_ref, feat_ref, fusw_ref, gb_ref, fc1w_ref,
                fc2w_ref, acc_ref, out_ref):
    mask = _mask_of(feat_ref[...])
    yf = _yf(fts1_ref, fts2_ref, fusw_ref)
    sc, sh = _bn_params(acc_ref[0, :], acc_ref[1, :], CNTF,
                        gb_ref[0, :], gb_ref[1, :])
    h = jnp.maximum(yf * sc + sh, 0.0).reshape(BB, N, 128)
    h = h * jnp.transpose(mask, (0, 2, 1))
    counts = jnp.maximum(jnp.sum(mask, axis=2), 1.0)
    pooled = jnp.sum(h, axis=1) / counts
    a = jnp.maximum(_mm(pooled, fc1w_ref[...]) + gb_ref[2, :], 0.0)
    out_ref[...] = _mm(a, fc2w_ref[...]) + gb_ref[3, 0:16]


def _bcast2(shape):
    return pl.BlockSpec(shape, lambda s, b: (0,) * len(shape))


def _bblk2(ch):
    return pl.BlockSpec((BB, ch, N), lambda s, b: (b, 0, 0))


def _bcast1(shape):
    return pl.BlockSpec(shape, lambda b: (0,) * len(shape))


def _bblk1(ch):
    return pl.BlockSpec((BB, ch, N), lambda b: (b, 0, 0))


def kernel(points, features, bn_fts_g, bn_fts_b, ec1_W0, ec1_W1, ec1_W2,
           ec1_g0, ec1_b0, ec1_g1, ec1_b1, ec1_g2, ec1_b2,
           ec2_W0, ec2_W1, ec2_W2, ec2_g0, ec2_b0, ec2_g1, ec2_b1,
           ec2_g2, ec2_b2, ec2_Wsc, ec2_sc_g, ec2_sc_b,
           fus_W, fus_g, fus_b, fc1_W, fc1_b, fc2_W, fc2_b):
    f32 = jnp.float32
    pts8 = jnp.concatenate(
        [points, jnp.zeros((B, 5, N), f32)], axis=1)

    ec1_w = jnp.zeros((96, 64), f32)
    ec1_w = ec1_w.at[0:32, 0:64].set(ec1_W0)
    ec1_w = ec1_w.at[32:64, 0:32].set(ec1_W1)
    ec1_w = ec1_w.at[64:96, 0:32].set(ec1_W2)
    ec1_gb = jnp.stack([ec1_g0, ec1_b0, ec1_g1, ec1_b1, ec1_g2, ec1_b2])
    g0 = bn_fts_g.reshape(1, C1)
    b0 = bn_fts_b.reshape(1, C1)

    ec1_in = [pts8, features, g0, b0, ec1_w, ec1_gb]
    acc1 = pl.pallas_call(
        _ec1_stats,
        grid=(4, NB),
        in_specs=[_bblk2(8), _bblk2(C1), _bcast2((1, C1)), _bcast2((1, C1)),
                  _bcast2((96, 64)), _bcast2((6, C1))],
        out_specs=_bcast2((8, C1)),
        out_shape=jax.ShapeDtypeStruct((8, C1), f32),
        scratch_shapes=[pltpu.VMEM((8, B * N), jnp.int32)],
    )(*ec1_in)

    fts1 = pl.pallas_call(
        _ec1_apply,
        grid=(NB,),
        in_specs=[_bblk1(8), _bblk1(C1), _bcast1((1, C1)), _bcast1((1, C1)),
                  _bcast1((96, 64)), _bcast1((6, C1)), _bcast1((8, C1))],
        out_specs=_bblk1(C1),
        out_shape=jax.ShapeDtypeStruct((B, C1, N), f32),
    )(*ec1_in, acc1)

    ec2_w = jnp.concatenate([ec2_W0, ec2_W1, ec2_W2], axis=0)
    ec2_gb = jnp.stack([ec2_g0, ec2_b0, ec2_g1, ec2_b1, ec2_g2, ec2_b2,
                        ec2_sc_g, ec2_sc_b])

    ec2_in = [fts1, features, ec2_Wsc, ec2_w, ec2_gb]
    acc2 = pl.pallas_call(
        _ec2_stats,
        grid=(4, NB),
        in_specs=[_bblk2(C1), _bblk2(C1), _bcast2((C2, C1)),
                  _bcast2((192, C2)), _bcast2((8, C2))],
        out_specs=_bcast2((8, C2)),
        out_shape=jax.ShapeDtypeStruct((8, C2), f32),
        scratch_shapes=[pltpu.VMEM((8, B * N), jnp.int32)],
    )(*ec2_in)

    fts2 = pl.pallas_call(
        _ec2_apply,
        grid=(NB,),
        in_specs=[_bblk1(C1), _bblk1(C1), _bcast1((C2, C1)),
                  _bcast1((192, C2)), _bcast1((8, C2)), _bcast1((8, C2))],
        out_specs=_bblk1(C2),
        out_shape=jax.ShapeDtypeStruct((B, C2, N), f32),
    )(*ec2_in, acc2)

    acch = pl.pallas_call(
        _head_stats,
        grid=(NB,),
        in_specs=[_bblk1(C1), _bblk1(C2), _bcast1((128, 96))],
        out_specs=_bcast1((2, 128)),
        out_shape=jax.ShapeDtypeStruct((2, 128), f32),
    )(fts1, fts2, fus_W)

    fc2_pad = jnp.zeros((16, 128), f32).at[0:10, :].set(fc2_W)
    head_gb = jnp.stack([fus_g, fus_b, fc1_b,
                         jnp.zeros((128,), f32).at[0:10].set(fc2_b)])

    out16 = pl.pallas_call(
        _head_apply,
        grid=(NB,),
        in_specs=[_bblk1(C1), _bblk1(C2), _bblk1(C1), _bcast1((128, 96)),
                  _bcast1((4, 128)), _bcast1((128, 128)), _bcast1((16, 128)),
                  _bcast1((2, 128))],
        out_specs=pl.BlockSpec((BB, 16), lambda b: (b, 0)),
        out_shape=jax.ShapeDtypeStruct((B, 16), f32),
    )(fts1, fts2, features, fus_W, head_gb, fc1_W, fc2_pad, acch)

    return out16[:, 0:10]


# speedup vs baseline: 8.3977x; 1.3049x over previous
import jax
import jax.numpy as jnp
from jax.experimental import pallas as pl
from jax.experimental.pallas import tpu as pltpu

B = 256
N = 128
K = 7
C1 = 32
C2 = 64
BB = 8
NB = B // BB
NK = N * K
EPS = 1e-5
NEG = -3.0e38
CNTF = float(B * N)
CNTK = float(B * N * K)


def _bn_params(s, ss, cnt, g, b):
    m = s / cnt
    v = ss / cnt - m * m
    scale = g * jax.lax.rsqrt(v + EPS)
    shift = b - m * scale
    return scale, shift


def _mask_of(f):
    return (jnp.sum(jnp.abs(f), axis=1, keepdims=True) != 0.0).astype(jnp.float32)


def _knn_sel(pts):
    dg = jax.lax.broadcasted_iota(jnp.int32, (N, N), 0) == \
        jax.lax.broadcasted_iota(jnp.int32, (N, N), 1)
    rows = []
    for i in range(BB):
        p = pts[i]
        pt = p.T
        g = jax.lax.dot_general(pt, p, (((1,), (0,)), ((), ())),
                                preferred_element_type=jnp.float32)
        xx = jnp.sum(pt * pt, axis=1, keepdims=True)
        rows.append(jnp.where(dg, NEG, 2.0 * g - xx - xx.T))
    pd = jnp.concatenate(rows, axis=0)
    sels = []
    for _ in range(K):
        m = jnp.max(pd, axis=1, keepdims=True)
        sel = pd >= m
        sels.append(sel)
        pd = jnp.where(sel, NEG, pd)
    return sels


def _sel_pack(sels):
    bits = sels[0].astype(jnp.int32)
    for j in range(1, K):
        bits = bits + (sels[j].astype(jnp.int32) << j)
    return bits


def _ohs_from_sels(sels):
    return [jnp.concatenate(
        [sels[j][i * N:(i + 1) * N, :].astype(jnp.float32) for j in range(K)],
        axis=0) for i in range(BB)]


def _ohs_from_bits(bits):
    outs = []
    for i in range(BB):
        bi = bits[i * N:(i + 1) * N, :]
        outs.append(jnp.concatenate(
            [((bi >> j) & 1).astype(jnp.float32) for j in range(K)], axis=0))
    return outs


def _mm(x, w):
    return jax.lax.dot_general(x, w, (((1,), (1,)), ((), ())),
                               preferred_element_type=jnp.float32)


def _edge_conv1(ftsT, ohs, w):
    C = ftsT.shape[-1]
    wd = w[:, 0:C] - w[:, C:2 * C]
    wb = w[:, C:2 * C]
    outs = []
    for i in range(BB):
        base = _mm(ftsT[i], wd)
        gsrc = _mm(ftsT[i], wb)
        nb = jax.lax.dot_general(ohs[i], gsrc, (((1,), (0,)), ((), ())),
                                 preferred_element_type=jnp.float32)
        outs.append(nb + jnp.concatenate([base] * K, axis=0))
    return jnp.concatenate(outs, axis=0)


def _acc(ref, row, y):
    ones = jnp.ones((1, y.shape[0]), jnp.float32)
    s = jax.lax.dot_general(ones, y, (((1,), (0,)), ((), ())),
                            preferred_element_type=jnp.float32)
    g = jax.lax.dot_general(y, y, (((0,), (0,)), ((), ())),
                            preferred_element_type=jnp.float32)
    c = y.shape[1]
    eye = (jax.lax.broadcasted_iota(jnp.int32, (c, c), 0) ==
           jax.lax.broadcasted_iota(jnp.int32, (c, c), 1))
    ss = jnp.sum(jnp.where(eye, g, 0.0), axis=0)
    ref[row, :] += s[0, :]
    ref[row + 1, :] += ss


def _ec1_fn(f, mask, acc, g0, b0):
    sc0, sh0 = _bn_params(acc[0, :], acc[1, :], CNTF, g0[0, :], b0[0, :])
    return (f * sc0[None, :, None] + sh0[None, :, None]) * mask


def _ec1_stats(points_ref, feat_ref, g0_ref, b0_ref, w_ref, gb_ref,
               acc_ref, idx_s):
    s = pl.program_id(0)
    blk = pl.program_id(1)

    @pl.when(jnp.logical_and(s == 0, blk == 0))
    def _zero():
        acc_ref[...] = jnp.zeros_like(acc_ref)

    f = feat_ref[...]
    mask = _mask_of(f)

    @pl.when(s == 0)
    def _s0():
        pts = points_ref[...] * mask + (1.0 - mask) * 1e9
        idx_s[pl.ds(blk * BB * N, BB * N), :] = _sel_pack(_knn_sel(pts))
        acc_ref[0, :] += jnp.sum(f, axis=(0, 2))
        acc_ref[1, :] += jnp.sum(f * f, axis=(0, 2))

    @pl.when(s > 0)
    def _rest():
        fn = _ec1_fn(f, mask, acc_ref, g0_ref, b0_ref)
        ftsT = jnp.transpose(fn, (0, 2, 1))
        ohs = _ohs_from_bits(idx_s[pl.ds(blk * BB * N, BB * N), :])
        y = _edge_conv1(ftsT, ohs, w_ref[0:32, 0:64])

        @pl.when(s == 1)
        def _():
            _acc(acc_ref, 2, y)

        @pl.when(s > 1)
        def _d1():
            sc, sh = _bn_params(acc_ref[2, :], acc_ref[3, :], CNTK,
                                gb_ref[0, :], gb_ref[1, :])
            y2 = _mm(jnp.maximum(y * sc + sh, 0.0), w_ref[32:64, 0:32])

            @pl.when(s == 2)
            def _():
                _acc(acc_ref, 4, y2)

            @pl.when(s == 3)
            def _d2():
                sc2, sh2 = _bn_params(acc_ref[4, :], acc_ref[5, :], CNTK,
                                      gb_ref[2, :], gb_ref[3, :])
                y3 = _mm(jnp.maximum(y2 * sc2 + sh2, 0.0), w_ref[64:96, 0:32])
                _acc(acc_ref, 6, y3)


def _ec1_apply(points_ref, feat_ref, g0_ref, b0_ref, w_ref, gb_ref, acc_ref,
               out_ref):
    f = feat_ref[...]
    mask = _mask_of(f)
    pts = points_ref[...] * mask + (1.0 - mask) * 1e9
    ohs = _ohs_from_sels(_knn_sel(pts))
    fn = _ec1_fn(f, mask, acc_ref, g0_ref, b0_ref)
    ftsT = jnp.transpose(fn, (0, 2, 1))
    y = _edge_conv1(ftsT, ohs, w_ref[0:32, 0:64])
    sc, sh = _bn_params(acc_ref[2, :], acc_ref[3, :], CNTK,
                        gb_ref[0, :], gb_ref[1, :])
    y2 = _mm(jnp.maximum(y * sc + sh, 0.0), w_ref[32:64, 0:32])
    sc2, sh2 = _bn_params(acc_ref[4, :], acc_ref[5, :], CNTK,
                          gb_ref[2, :], gb_ref[3, :])
    y3 = _mm(jnp.maximum(y2 * sc2 + sh2, 0.0), w_ref[64:96, 0:32])
    sc3, sh3 = _bn_params(acc_ref[6, :], acc_ref[7, :], CNTK,
                          gb_ref[4, :], gb_ref[5, :])
    z3 = jnp.maximum(y3 * sc3 + sh3, 0.0)
    zm = jnp.mean(z3.reshape(BB, K, N, C1), axis=1)
    o = jnp.maximum(ftsT + zm, 0.0)
    out_ref[...] = jnp.transpose(o, (0, 2, 1)) * mask


def _ec2_stats(fts1_ref, feat_ref, wsc_ref, w_ref, gb_ref, acc_ref, idx_s):
    s = pl.program_id(0)
    blk = pl.program_id(1)

    @pl.when(jnp.logical_and(s == 0, blk == 0))
    def _zero():
        acc_ref[...] = jnp.zeros_like(acc_ref)

    f1 = fts1_ref[...]
    mask = _mask_of(feat_ref[...])
    f1T = jnp.transpose(f1, (0, 2, 1)).reshape(BB * N, C1)

    @pl.when(s == 0)
    def _s0():
        pts = f1 + (1.0 - mask) * 1e9
        idx_s[pl.ds(blk * BB * N, BB * N), :] = _sel_pack(_knn_sel(pts))
        _acc(acc_ref, 0, _mm(f1T, wsc_ref[...]))

    @pl.when(s > 0)
    def _rest():
        ftsT = f1T.reshape(BB, N, C1)
        ohs = _ohs_from_bits(idx_s[pl.ds(blk * BB * N, BB * N), :])
        y = _edge_conv1(ftsT, ohs, w_ref[0:64, :])

        @pl.when(s == 1)
        def _():
            _acc(acc_ref, 2, y)

        @pl.when(s > 1)
        def _d1():
            sc, sh = _bn_params(acc_ref[2, :], acc_ref[3, :], CNTK,
                                gb_ref[0, :], gb_ref[1, :])
            y2 = _mm(jnp.maximum(y * sc + sh, 0.0), w_ref[64:128, :])

            @pl.when(s == 2)
            def _():
                _acc(acc_ref, 4, y2)

            @pl.when(s == 3)
            def _d2():
                sc2, sh2 = _bn_params(acc_ref[4, :], acc_ref[5, :], CNTK,
                                      gb_ref[2, :], gb_ref[3, :])
                y3 = _mm(jnp.maximum(y2 * sc2 + sh2, 0.0), w_ref[128:192, :])
                _acc(acc_ref, 6, y3)


def _ec2_apply(fts1_ref, feat_ref, wsc_ref, w_ref, gb_ref, acc_ref, out_ref):
    f1 = fts1_ref[...]
    mask = _mask_of(feat_ref[...])
    f1T = jnp.transpose(f1, (0, 2, 1)).reshape(BB * N, C1)
    pts = f1 + (1.0 - mask) * 1e9
    ohs = _ohs_from_sels(_knn_sel(pts))
    ftsT = f1T.reshape(BB, N, C1)
    y = _edge_conv1(ftsT, ohs, w_ref[0:64, :])
    sc, sh = _bn_params(acc_ref[2, :], acc_ref[3, :], CNTK,
                        gb_ref[0, :], gb_ref[1, :])
    y2 = _mm(jnp.maximum(y * sc + sh, 0.0), w_ref[64:128, :])
    sc2, sh2 = _bn_params(acc_ref[4, :], acc_ref[5, :], CNTK,
                          gb_ref[2, :], gb_ref[3, :])
    y3 = _mm(jnp.maximum(y2 * sc2 + sh2, 0.0), w_ref[128:192, :])
    sc3, sh3 = _bn_params(acc_ref[6, :], acc_ref[7, :], CNTK,
                          gb_ref[4, :], gb_ref[5, :])
    z3 = jnp.maximum(y3 * sc3 + sh3, 0.0)
    zm = jnp.mean(z3.reshape(BB, K, N, C2), axis=1)
    scp, shp = _bn_params(acc_ref[0, :], acc_ref[1, :], CNTF,
                          gb_ref[6, :], gb_ref[7, :])
    scv = _mm(f1T, wsc_ref[...]).reshape(BB, N, C2) * scp + shp
    o = jnp.maximum(scv + zm, 0.0)
    out_ref[...] = jnp.transpose(o, (0, 2, 1)) * mask


def _yf(fts1_ref, fts2_ref, fusw_ref):
    cat = jnp.concatenate(
        [jnp.transpose(fts1_ref[...], (0, 2, 1)),
         jnp.transpose(fts2_ref[...], (0, 2, 1))], axis=2)
    return _mm(cat.reshape(BB * N, 96), fusw_ref[...])


def _head_stats(fts1_ref, fts2_ref, fusw_ref, acc_ref):
    blk = pl.program_id(0)

    @pl.when(blk == 0)
    def _zero():
        acc_ref[...] = jnp.zeros_like(acc_ref)

    _acc(acc_ref, 0, _yf(fts1_ref, fts2_ref, fusw_ref))


def _head_apply(fts1_ref, fts2_ref, feat_ref, fusw_ref, gb_ref, fc1w_ref,
                fc2w_ref, acc_ref, out_ref):
    mask = _mask_of(feat_ref[...])
    yf = _yf(fts1_ref, fts2_ref, fusw_ref)
    sc, sh = _bn_params(acc_ref[0, :], acc_ref[1, :], CNTF,
                        gb_ref[0, :], gb_ref[1, :])
    h = jnp.maximum(yf * sc + sh, 0.0).reshape(BB, N, 128)
    h = h * jnp.transpose(mask, (0, 2, 1))
    counts = jnp.maximum(jnp.sum(mask, axis=2), 1.0)
    pooled = jnp.sum(h, axis=1) / counts
    a = jnp.maximum(_mm(pooled, fc1w_ref[...]) + gb_ref[2, :], 0.0)
    out_ref[...] = _mm(a, fc2w_ref[...]) + gb_ref[3, 0:16]


def _bcast2(shape):
    return pl.BlockSpec(shape, lambda s, b: (0,) * len(shape))


def _bblk2(ch):
    return pl.BlockSpec((BB, ch, N), lambda s, b: (b, 0, 0))


def _bcast1(shape):
    return pl.BlockSpec(shape, lambda b: (0,) * len(shape))


def _bblk1(ch):
    return pl.BlockSpec((BB, ch, N), lambda b: (b, 0, 0))


def kernel(points, features, bn_fts_g, bn_fts_b, ec1_W0, ec1_W1, ec1_W2,
           ec1_g0, ec1_b0, ec1_g1, ec1_b1, ec1_g2, ec1_b2,
           ec2_W0, ec2_W1, ec2_W2, ec2_g0, ec2_b0, ec2_g1, ec2_b1,
           ec2_g2, ec2_b2, ec2_Wsc, ec2_sc_g, ec2_sc_b,
           fus_W, fus_g, fus_b, fc1_W, fc1_b, fc2_W, fc2_b):
    f32 = jnp.float32
    pts8 = jnp.concatenate(
        [points, jnp.zeros((B, 5, N), f32)], axis=1)

    ec1_w = jnp.zeros((96, 64), f32)
    ec1_w = ec1_w.at[0:32, 0:64].set(ec1_W0)
    ec1_w = ec1_w.at[32:64, 0:32].set(ec1_W1)
    ec1_w = ec1_w.at[64:96, 0:32].set(ec1_W2)
    ec1_gb = jnp.stack([ec1_g0, ec1_b0, ec1_g1, ec1_b1, ec1_g2, ec1_b2])
    g0 = bn_fts_g.reshape(1, C1)
    b0 = bn_fts_b.reshape(1, C1)

    ec1_in = [pts8, features, g0, b0, ec1_w, ec1_gb]
    acc1 = pl.pallas_call(
        _ec1_stats,
        grid=(4, NB),
        in_specs=[_bblk2(8), _bblk2(C1), _bcast2((1, C1)), _bcast2((1, C1)),
                  _bcast2((96, 64)), _bcast2((6, C1))],
        out_specs=_bcast2((8, C1)),
        out_shape=jax.ShapeDtypeStruct((8, C1), f32),
        scratch_shapes=[pltpu.VMEM((B * N, N), jnp.int32)],
    )(*ec1_in)

    fts1 = pl.pallas_call(
        _ec1_apply,
        grid=(NB,),
        in_specs=[_bblk1(8), _bblk1(C1), _bcast1((1, C1)), _bcast1((1, C1)),
                  _bcast1((96, 64)), _bcast1((6, C1)), _bcast1((8, C1))],
        out_specs=_bblk1(C1),
        out_shape=jax.ShapeDtypeStruct((B, C1, N), f32),
    )(*ec1_in, acc1)

    ec2_w = jnp.concatenate([ec2_W0, ec2_W1, ec2_W2], axis=0)
    ec2_gb = jnp.stack([ec2_g0, ec2_b0, ec2_g1, ec2_b1, ec2_g2, ec2_b2,
                        ec2_sc_g, ec2_sc_b])

    ec2_in = [fts1, features, ec2_Wsc, ec2_w, ec2_gb]
    acc2 = pl.pallas_call(
        _ec2_stats,
        grid=(4, NB),
        in_specs=[_bblk2(C1), _bblk2(C1), _bcast2((C2, C1)),
                  _bcast2((192, C2)), _bcast2((8, C2))],
        out_specs=_bcast2((8, C2)),
        out_shape=jax.ShapeDtypeStruct((8, C2), f32),
        scratch_shapes=[pltpu.VMEM((B * N, N), jnp.int32)],
    )(*ec2_in)

    fts2 = pl.pallas_call(
        _ec2_apply,
        grid=(NB,),
        in_specs=[_bblk1(C1), _bblk1(C1), _bcast1((C2, C1)),
                  _bcast1((192, C2)), _bcast1((8, C2)), _bcast1((8, C2))],
        out_specs=_bblk1(C2),
        out_shape=jax.ShapeDtypeStruct((B, C2, N), f32),
    )(*ec2_in, acc2)

    acch = pl.pallas_call(
        _head_stats,
        grid=(NB,),
        in_specs=[_bblk1(C1), _bblk1(C2), _bcast1((128, 96))],
        out_specs=_bcast1((2, 128)),
        out_shape=jax.ShapeDtypeStruct((2, 128), f32),
    )(fts1, fts2, fus_W)

    fc2_pad = jnp.zeros((16, 128), f32).at[0:10, :].set(fc2_W)
    head_gb = jnp.stack([fus_g, fus_b, fc1_b,
                         jnp.zeros((128,), f32).at[0:10].set(fc2_b)])

    out16 = pl.pallas_call(
        _head_apply,
        grid=(NB,),
        in_specs=[_bblk1(C1), _bblk1(C2), _bblk1(C1), _bcast1((128, 96)),
                  _bcast1((4, 128)), _bcast1((128, 128)), _bcast1((16, 128)),
                  _bcast1((2, 128))],
        out_specs=pl.BlockSpec((BB, 16), lambda b: (b, 0)),
        out_shape=jax.ShapeDtypeStruct((B, 16), f32),
    )(fts1, fts2, features, fus_W, head_gb, fc1_W, fc2_pad, acch)

    return out16[:, 0:10]
